# baseline (speedup 1.0000x reference)
import functools
import math

import jax
import jax.numpy as jnp
import numpy as np
from jax import lax
from jax.experimental import pallas as pl
from jax.experimental.pallas import tpu as pltpu
from jax.experimental.pallas import tpu_sc as plsc

_N = 10000
_E = 320000
_G = 16
_NC, _NS = 2, 16
_NW = _NC * _NS
_EPW = _E // _NW
_CH = 80
_NCHUNK = _EPW // _CH
_RPT = _N // _NS
_INV_SQRT_NN = 1.0 / math.sqrt(32.0)

_MESH = dict(core_axis_name="c", subcore_axis_name="s",
             num_cores=_NC, num_subcores=_NS)


_NBUF = 5
_NSUP = _NCHUNK // _NBUF


def _geom_scratch():
    refs = []
    for _ in range(_NBUF):
        refs += [
            pltpu.VMEM((_CH,), jnp.int32),
            pltpu.VMEM((_CH,), jnp.int32),
            pltpu.VMEM((_CH, 16), jnp.float32),
            pltpu.VMEM((_CH, 16), jnp.float32),
            pltpu.SemaphoreType.DMA,
        ]
    return refs


@functools.partial(
    pl.kernel,
    out_type=jax.ShapeDtypeStruct((_E, 16), jnp.float32),
    mesh=plsc.VectorSubcoreMesh(**_MESH),
    scratch_types=_geom_scratch(),
    compiler_params=pltpu.CompilerParams(use_tc_tiling_on_sc=False, needs_layout_passes=False),
)
def _geom(pos_hbm, src_hbm, dst_hbm, vec_hbm, *scr):
    cid = lax.axis_index("c")
    sid = lax.axis_index("s")
    wid = cid * _NS + sid
    slots = [scr[5 * b:5 * b + 5] for b in range(_NBUF)]

    def super_it(g, carry):
        base = wid * _EPW + g * (_NBUF * _CH)
        di = []
        for b, (sidx, didx, ps, pd, sem) in enumerate(slots):
            off = base + b * _CH
            di.append((
                pltpu.async_copy(src_hbm.at[pl.ds(off, _CH)], sidx, sem),
                pltpu.async_copy(dst_hbm.at[pl.ds(off, _CH)], didx, sem),
            ))
        dg = []
        for b, (sidx, didx, ps, pd, sem) in enumerate(slots):
            di[b][0].wait()
            di[b][1].wait()
            dg.append((
                pltpu.async_copy(pos_hbm.at[sidx], ps, sem),
                pltpu.async_copy(pos_hbm.at[didx], pd, sem),
            ))
        dv = []
        for b, (sidx, didx, ps, pd, sem) in enumerate(slots):
            off = base + b * _CH
            dg[b][0].wait()
            dg[b][1].wait()

            @plsc.parallel_loop(0, _CH, unroll=4)
            def row(i, ps=ps, pd=pd):
                ps[i, :] = ps[i, :] - pd[i, :]

            dv.append(pltpu.async_copy(ps, vec_hbm.at[pl.ds(off, _CH)], sem))
        for d in dv:
            d.wait()
        return carry

    lax.fori_loop(0, _NSUP, super_it, 0)


def _pack_tc(a, b):
    ua = jax.lax.bitcast_convert_type(a, jnp.uint32)
    ub = jax.lax.bitcast_convert_type(b, jnp.uint32)
    la = (ua + 0x7FFF + ((ua >> 16) & 1)) >> 16
    hb = (ub + 0x7FFF + ((ub >> 16) & 1)) & jnp.uint32(0xFFFF0000)
    return jax.lax.bitcast_convert_type(la | hb, jnp.int32)


def _unpack_lo(v):
    return plsc.bitcast(v << 16, jnp.float32)


def _unpack_hi(v):
    return plsc.bitcast(v & jnp.int32(-65536), jnp.float32)


@functools.lru_cache(maxsize=None)
def _make_msg(dp):
    hp = dp // 2
    nbuf = 2 if dp == 128 else 5
    nsup = _NCHUNK // nbuf
    ntail = _NCHUNK - nsup * nbuf
    scratch = []
    for _ in range(nbuf):
        scratch += [
            pltpu.VMEM((_CH,), jnp.int32),
            pltpu.VMEM((_CH,), jnp.int32),
            pltpu.VMEM((_CH, hp), jnp.int32),
            pltpu.VMEM((_CH, hp), jnp.int32),
            pltpu.VMEM((_CH, dp), jnp.float32),
            pltpu.SemaphoreType.DMA,
        ]
    scratch.append(pltpu.VMEM_SHARED((_N, dp), jnp.float32))

    @functools.partial(
        pl.kernel,
        out_type=jax.ShapeDtypeStruct((2 * _N, dp), jnp.float32),
        mesh=plsc.VectorSubcoreMesh(**_MESH),
        scratch_types=scratch,
        compiler_params=pltpu.CompilerParams(use_tc_tiling_on_sc=False, needs_layout_passes=False),
    )
    def msg(x_hbm, c_hbm, src_hbm, dst_hbm, out_hbm, *scr):
        cid = lax.axis_index("c")
        sid = lax.axis_index("s")
        wid = cid * _NS + sid
        slots = [scr[6 * b:6 * b + 6] for b in range(nbuf)]
        aggsh = scr[-1]
        zv = jnp.zeros((16,), jnp.float32)

        def mul_rows(xg, cg, m):
            @plsc.parallel_loop(0, _CH, unroll=2)
            def mrow(i):
                for j in range(dp // 32):
                    sl = pl.ds(j * 16, 16)
                    pi = xg[i, sl]
                    ci = cg[i, sl]
                    m[i, sl] = _unpack_lo(pi) * _unpack_lo(ci)
                    m[i, pl.ds(hp + j * 16, 16)] = (
                        _unpack_hi(pi) * _unpack_hi(ci))

        m0 = slots[0][4]

        @plsc.parallel_loop(0, _CH, unroll=4)
        def zrow(i):
            for j in range(dp // 16):
                m0[i, pl.ds(j * 16, 16)] = zv

        nfull = _RPT // _CH
        rem = _RPT - nfull * _CH

        def zslab(t, c2):
            pltpu.sync_copy(m0, aggsh.at[pl.ds(sid * _RPT + t * _CH, _CH)])
            return c2

        lax.fori_loop(0, nfull, zslab, 0)
        if rem:
            pltpu.sync_copy(m0.at[pl.ds(0, rem)],
                            aggsh.at[pl.ds(sid * _RPT + nfull * _CH, rem)])
        plsc.subcore_barrier()

        def super_it(g, carry):
            base = wid * _EPW + g * (nbuf * _CH)
            di = []
            for b, (sidx, didx, xg, cg, m, sem) in enumerate(slots):
                off = base + b * _CH
                di.append((
                    pltpu.async_copy(src_hbm.at[pl.ds(off, _CH)], sidx, sem),
                    pltpu.async_copy(dst_hbm.at[pl.ds(off, _CH)], didx, sem),
                ))
            dg = []
            for b, (sidx, didx, xg, cg, m, sem) in enumerate(slots):
                off = base + b * _CH
                di[b][0].wait()
                di[b][1].wait()
                dg.append((
                    pltpu.async_copy(x_hbm.at[sidx], xg, sem),
                    pltpu.async_copy(c_hbm.at[pl.ds(off, _CH)], cg, sem),
                ))
            ds = []
            for b, (sidx, didx, xg, cg, m, sem) in enumerate(slots):
                dg[b][0].wait()
                dg[b][1].wait()
                mul_rows(xg, cg, m)
                ds.append(pltpu.async_copy(m, aggsh.at[didx], sem, add=True))
            for d in ds:
                d.wait()
            return carry

        lax.fori_loop(0, nsup, super_it, 0)

        def tail(k, carry):
            off = wid * _EPW + (nsup * nbuf + k) * _CH
            sidx, didx, xg, cg, m, sem = slots[0]
            pltpu.sync_copy(src_hbm.at[pl.ds(off, _CH)], sidx)
            pltpu.sync_copy(dst_hbm.at[pl.ds(off, _CH)], didx)
            pltpu.async_copy(x_hbm.at[sidx], xg, sem).wait()
            pltpu.sync_copy(c_hbm.at[pl.ds(off, _CH)], cg)
            mul_rows(xg, cg, m)
            pltpu.sync_copy(m, aggsh.at[didx], add=True)
            return carry

        if ntail:
            lax.fori_loop(0, ntail, tail, 0)
        plsc.subcore_barrier()
        pltpu.sync_copy(aggsh.at[pl.ds(sid * _RPT, _RPT)],
                        out_hbm.at[pl.ds(cid * _N + sid * _RPT, _RPT)])

    return msg


_BE = 2000
_S3 = math.sqrt(3.0)
_S5 = math.sqrt(5.0)
_S15 = math.sqrt(15.0)
_RB_VALS = np.linspace(0.0, 5.0, 12)[1:-1]
_RB_STEP = float(_RB_VALS[1] - _RB_VALS[0])
_RB_C = 1.14136 * (math.e ** 2)
_SQRT_NB = math.sqrt(10.0)


def _coef_body(dps, *refs):
    vec_ref = refs[0]
    wrefs = refs[1:1 + 4 * len(dps)]
    orefs = refs[1 + 4 * len(dps):]

    v = vec_ref[...]
    x = v[:, 0:1]
    y = v[:, 1:2]
    z = v[:, 2:3]
    r = jnp.sqrt(x * x + y * y + z * z + 1e-9)
    inv = 1.0 / r
    ux, uy, uz = x * inv, y * inv, z * inv
    one = jnp.ones_like(ux)
    zero = jnp.zeros_like(ux)
    sh = jnp.concatenate([
        one,
        _S3 * ux, _S3 * uy, _S3 * uz,
        _S15 * ux * uz, _S15 * ux * uy,
        _S5 * (uy * uy - 0.5 * (ux * ux + uz * uz)),
        _S15 * uy * uz, 0.5 * _S15 * (uz * uz - ux * ux),
        zero, zero, zero, zero, zero, zero, zero,
    ], axis=1)

    kk = lax.broadcasted_iota(jnp.int32, (1, 10), 1).astype(jnp.float32)
    vals = (kk + 1.0) * _RB_STEP
    diff = (r - vals) * (1.0 / _RB_STEP)
    inside = jnp.abs(diff) < 1.0
    denom = jnp.maximum(jnp.where(inside, 1.0 - diff * diff, 1.0), 1e-6)
    emb = jnp.where(inside, _RB_C * jnp.exp(-1.0 / denom), 0.0) * _SQRT_NB
    emb = jnp.concatenate([emb, jnp.zeros_like(v[:, 0:6])], axis=1)

    for l in range(len(dps)):
        w1, b1, w2, wsh = wrefs[4 * l:4 * l + 4]
        h = jnp.dot(emb, w1[...], preferred_element_type=jnp.float32) + b1[...]
        h = h * jax.nn.sigmoid(h)
        c = (jnp.dot(h, w2[...], preferred_element_type=jnp.float32)
             * jnp.dot(sh, wsh[...], preferred_element_type=jnp.float32))
        hp = dps[l] // 2
        orefs[l][...] = _pack_tc(c[:, :hp], c[:, hp:])


def _coef(vec, wlist, dps):
    flat_w = [w for tup in wlist for w in tup]
    full = lambda shape: pl.BlockSpec(shape, lambda i: (0, 0))
    return pl.pallas_call(
        functools.partial(_coef_body, tuple(dps)),
        grid=(_E // _BE,),
        in_specs=[pl.BlockSpec((_BE, 16), lambda i: (i, 0))]
        + [full(w.shape) for w in flat_w],
        out_specs=[pl.BlockSpec((_BE, dp // 2), lambda i: (i, 0))
                   for dp in dps],
        out_shape=[jax.ShapeDtypeStruct((_E, dp // 2), jnp.int32)
                   for dp in dps],
    )(vec, *flat_w)


_BN = 2000


def _node_body(last, dpo, x_ref, na_ref, agg_ref, wsc_ref, wmsg_ref, o_ref,
               op_ref=None):
    xn = x_ref[...] * na_ref[...]
    sc = jnp.dot(xn, wsc_ref[...], preferred_element_type=jnp.float32)
    a = (agg_ref[0] + agg_ref[1]) * _INV_SQRT_NN
    out = sc + jnp.dot(a, wmsg_ref[...], preferred_element_type=jnp.float32)
    if not last:
        out = out * jax.nn.sigmoid(out)
        op_ref[...] = _pack_tc(out[:, :dpo // 2], out[:, dpo // 2:])
    o_ref[...] = out


def _node(xp, na, agg2, wsc, wmsg, last, dpo):
    dpi = xp.shape[1]
    agg3 = agg2.reshape(2, _N, dpi)
    out_specs = [pl.BlockSpec((_BN, dpo), lambda i: (i, 0))]
    out_shape = [jax.ShapeDtypeStruct((_N, dpo), jnp.float32)]
    if not last:
        out_specs.append(pl.BlockSpec((_BN, dpo // 2), lambda i: (i, 0)))
        out_shape.append(jax.ShapeDtypeStruct((_N, dpo // 2), jnp.int32))
    res = pl.pallas_call(
        functools.partial(_node_body, last, dpo),
        grid=(_N // _BN,),
        in_specs=[
            pl.BlockSpec((_BN, dpi), lambda i: (i, 0)),
            pl.BlockSpec((_BN, 1), lambda i: (i, 0)),
            pl.BlockSpec((2, _BN, dpi), lambda i: (0, i, 0)),
            pl.BlockSpec(wsc.shape, lambda i: (0, 0)),
            pl.BlockSpec(wmsg.shape, lambda i: (0, 0)),
        ],
        out_specs=out_specs,
        out_shape=out_shape,
    )(xp, na, agg3, wsc, wmsg)
    return res if not last else (res[0], None)


def _pack_body(x_ref, o_ref):
    x = x_ref[...]
    o_ref[...] = _pack_tc(x[:, :64], x[:, 64:])


def _pack128(x):
    return pl.pallas_call(
        _pack_body,
        grid=(_N // _BN,),
        in_specs=[pl.BlockSpec((_BN, 128), lambda i: (i, 0))],
        out_specs=pl.BlockSpec((_BN, 64), lambda i: (i, 0)),
        out_shape=jax.ShapeDtypeStruct((_N, 64), jnp.int32),
    )(x)


def _mix_body(a_ref, b_ref, o_ref, op_ref):
    x = 0.5 * (a_ref[...] + b_ref[...])
    o_ref[...] = x
    op_ref[...] = _pack_tc(x[:, :64], x[:, 64:])


def _mix_pack(a, b):
    return pl.pallas_call(
        _mix_body,
        grid=(_N // _BN,),
        in_specs=[pl.BlockSpec((_BN, 128), lambda i: (i, 0)),
                  pl.BlockSpec((_BN, 128), lambda i: (i, 0))],
        out_specs=[pl.BlockSpec((_BN, 128), lambda i: (i, 0)),
                   pl.BlockSpec((_BN, 64), lambda i: (i, 0))],
        out_shape=[jax.ShapeDtypeStruct((_N, 128), jnp.float32),
                   jax.ShapeDtypeStruct((_N, 64), jnp.int32)],
    )(a, b)


def _norm_body(x_ref, b_ref, o_ref):
    xo = x_ref[...]
    bt = b_ref[...]
    ss = jnp.sum(xo * xo, axis=1, keepdims=True)
    gids = lax.broadcasted_iota(jnp.int32, (1, _G), 1)
    oh = (bt == gids).astype(jnp.float32)
    g = jnp.sum(oh * ss, axis=0, keepdims=True)
    f = jnp.sqrt(g + 1e-12)
    fb = jnp.sum(oh * f, axis=1, keepdims=True)
    o_ref[...] = xo / fb


def _norm(xo, bt):
    return pl.pallas_call(
        _norm_body,
        in_specs=[
            pl.BlockSpec((_N, 128), lambda: (0, 0)),
            pl.BlockSpec((_N, 1), lambda: (0, 0)),
        ],
        out_specs=pl.BlockSpec((_N, 128), lambda: (0, 0)),
        out_shape=jax.ShapeDtypeStruct((_N, 128), jnp.float32),
    )(xo, bt)


def _padw(w, r, c):
    return jnp.zeros((r, c), jnp.float32).at[:w.shape[0], :w.shape[1]].set(w)


def _run_net(pos16, xin, xin_p, na, src, dst, layers):
    vec = _geom(pos16, src, dst)
    dpis = [128, 64, 64, 64]
    dpos = [64, 64, 64, 128]
    wlist = [
        (
            _padw(p['W1'], 16, 64),
            p['b1'].reshape(1, 64),
            _padw(p['W2'], 64, dpis[l]),
            _padw(p['W_sh'], 16, dpis[l]),
        )
        for l, p in enumerate(layers)
    ]
    cs = _coef(vec, wlist, dpis)
    h, hpk = xin, xin_p
    for l, p in enumerate(layers):
        agg2 = _make_msg(dpis[l])(hpk, cs[l], src, dst)
        wsc = _padw(p['W_sc'], dpis[l], dpos[l])
        wmsg = _padw(p['W_msg'], dpis[l], dpos[l])
        h, hpk = _node(h, na, agg2, wsc, wmsg, last=(l == 3), dpo=dpos[l])
    return h


def kernel(x, pos, batch, edge_index, node_attr, x_final_state,
           pos_final_state, edge_index_final_state, params):
    f32 = jnp.float32
    src = edge_index[0].astype(jnp.int32)
    dst = edge_index[1].astype(jnp.int32)
    srcf = edge_index_final_state[0].astype(jnp.int32)
    dstf = edge_index_final_state[1].astype(jnp.int32)
    pos16 = jnp.zeros((_N, 16), f32).at[:, :3].set(pos)
    posf16 = jnp.zeros((_N, 16), f32).at[:, :3].set(pos_final_state)
    post16 = 0.5 * (pos16 + posf16)
    bt = batch.astype(jnp.int32).reshape(_N, 1)

    na = node_attr
    out_i = _norm(_run_net(pos16, x, _pack128(x), na, src, dst,
                           params['init']), bt)
    out_f = _norm(_run_net(posf16, x_final_state, _pack128(x_final_state),
                           na, srcf, dstf, params['final']), bt)
    x_ts, xts_p = _mix_pack(out_i, out_f)
    out_ts = _norm(_run_net(post16, x_ts, xts_p, na, src, dst,
                            params['interp']), bt)
    return out_ts


# speedup vs baseline: 1.6863x; 1.6863x over previous
import functools
import math

import jax
import jax.numpy as jnp
import numpy as np
from jax import lax
from jax.experimental import pallas as pl
from jax.experimental.pallas import tpu as pltpu
from jax.experimental.pallas import tpu_sc as plsc

_N = 10000
_E = 320000
_G = 16
_NC, _NS = 2, 16
_NW = _NC * _NS
_EPW = _E // _NW
_CH = 80
_NCHUNK = _EPW // _CH
_RPT = _N // _NS
_INV_SQRT_NN = 1.0 / math.sqrt(32.0)

_MESH = dict(core_axis_name="c", subcore_axis_name="s",
             num_cores=_NC, num_subcores=_NS)


_NBUF = 5
_NSUP = _NCHUNK // _NBUF


def _geom_scratch():
    refs = []
    for _ in range(_NBUF):
        refs += [
            pltpu.VMEM((_CH,), jnp.int32),
            pltpu.VMEM((_CH,), jnp.int32),
            pltpu.VMEM((_CH, 16), jnp.float32),
            pltpu.VMEM((_CH, 16), jnp.float32),
            pltpu.SemaphoreType.DMA,
        ]
    return refs


@functools.partial(
    pl.kernel,
    out_type=jax.ShapeDtypeStruct((_E, 16), jnp.float32),
    mesh=plsc.VectorSubcoreMesh(**_MESH),
    scratch_types=_geom_scratch(),
    compiler_params=pltpu.CompilerParams(use_tc_tiling_on_sc=False, needs_layout_passes=False),
)
def _geom(pos_hbm, src_hbm, dst_hbm, vec_hbm, *scr):
    cid = lax.axis_index("c")
    sid = lax.axis_index("s")
    wid = cid * _NS + sid
    slots = [scr[5 * b:5 * b + 5] for b in range(_NBUF)]

    def super_it(g, carry):
        base = wid * _EPW + g * (_NBUF * _CH)
        di = []
        for b, (sidx, didx, ps, pd, sem) in enumerate(slots):
            off = base + b * _CH
            di.append((
                pltpu.async_copy(src_hbm.at[pl.ds(off, _CH)], sidx, sem),
                pltpu.async_copy(dst_hbm.at[pl.ds(off, _CH)], didx, sem),
            ))
        dg = []
        for b, (sidx, didx, ps, pd, sem) in enumerate(slots):
            di[b][0].wait()
            di[b][1].wait()
            dg.append((
                pltpu.async_copy(pos_hbm.at[sidx], ps, sem),
                pltpu.async_copy(pos_hbm.at[didx], pd, sem),
            ))
        dv = []
        for b, (sidx, didx, ps, pd, sem) in enumerate(slots):
            off = base + b * _CH
            dg[b][0].wait()
            dg[b][1].wait()

            @plsc.parallel_loop(0, _CH, unroll=4)
            def row(i, ps=ps, pd=pd):
                ps[i, :] = ps[i, :] - pd[i, :]

            dv.append(pltpu.async_copy(ps, vec_hbm.at[pl.ds(off, _CH)], sem))
        for d in dv:
            d.wait()
        return carry

    lax.fori_loop(0, _NSUP, super_it, 0)


def _pack_tc(a, b):
    ua = jax.lax.bitcast_convert_type(a, jnp.uint32)
    ub = jax.lax.bitcast_convert_type(b, jnp.uint32)
    la = (ua + 0x8000) >> 16
    hb = (ub + 0x8000) & jnp.uint32(0xFFFF0000)
    return jax.lax.bitcast_convert_type(la | hb, jnp.int32)


def _unpack_lo(v):
    return plsc.bitcast(v << 16, jnp.float32)


def _unpack_hi(v):
    return plsc.bitcast(v & jnp.int32(-65536), jnp.float32)


@functools.lru_cache(maxsize=None)
def _make_msg(dp):
    hp = dp // 2
    nbuf = 2 if dp == 128 else 5
    nsup = _NCHUNK // nbuf
    ntail = _NCHUNK - nsup * nbuf
    scratch = []
    for _ in range(nbuf):
        scratch += [
            pltpu.VMEM((_CH,), jnp.int32),
            pltpu.VMEM((_CH,), jnp.int32),
            pltpu.VMEM((_CH, hp), jnp.int32),
            pltpu.VMEM((_CH, hp), jnp.int32),
            pltpu.VMEM((_CH, dp), jnp.float32),
            pltpu.SemaphoreType.DMA,
        ]
    scratch.append(pltpu.VMEM_SHARED((_N, dp), jnp.float32))

    @functools.partial(
        pl.kernel,
        out_type=jax.ShapeDtypeStruct((2 * _N, dp), jnp.float32),
        mesh=plsc.VectorSubcoreMesh(**_MESH),
        scratch_types=scratch,
        compiler_params=pltpu.CompilerParams(use_tc_tiling_on_sc=False, needs_layout_passes=False),
    )
    def msg(x_hbm, c_hbm, src_hbm, dst_hbm, out_hbm, *scr):
        cid = lax.axis_index("c")
        sid = lax.axis_index("s")
        wid = cid * _NS + sid
        slots = [scr[6 * b:6 * b + 6] for b in range(nbuf)]
        aggsh = scr[-1]
        zv = jnp.zeros((16,), jnp.float32)

        def mul_rows(xg, cg, m):
            @plsc.parallel_loop(0, _CH, unroll=2)
            def mrow(i):
                for j in range(dp // 32):
                    sl = pl.ds(j * 16, 16)
                    pi = xg[i, sl]
                    ci = cg[i, sl]
                    m[i, sl] = _unpack_lo(pi) * _unpack_lo(ci)
                    m[i, pl.ds(hp + j * 16, 16)] = (
                        _unpack_hi(pi) * _unpack_hi(ci))

        m0 = slots[0][4]

        @plsc.parallel_loop(0, _CH, unroll=4)
        def zrow(i):
            for j in range(dp // 16):
                m0[i, pl.ds(j * 16, 16)] = zv

        nfull = _RPT // _CH
        rem = _RPT - nfull * _CH

        def zslab(t, c2):
            pltpu.sync_copy(m0, aggsh.at[pl.ds(sid * _RPT + t * _CH, _CH)])
            return c2

        lax.fori_loop(0, nfull, zslab, 0)
        if rem:
            pltpu.sync_copy(m0.at[pl.ds(0, rem)],
                            aggsh.at[pl.ds(sid * _RPT + nfull * _CH, rem)])
        plsc.subcore_barrier()

        def super_it(g, carry):
            base = wid * _EPW + g * (nbuf * _CH)
            di = []
            for b, (sidx, didx, xg, cg, m, sem) in enumerate(slots):
                off = base + b * _CH
                di.append((
                    pltpu.async_copy(src_hbm.at[pl.ds(off, _CH)], sidx, sem),
                    pltpu.async_copy(dst_hbm.at[pl.ds(off, _CH)], didx, sem),
                ))
            dg = []
            for b, (sidx, didx, xg, cg, m, sem) in enumerate(slots):
                off = base + b * _CH
                di[b][0].wait()
                di[b][1].wait()
                dg.append((
                    pltpu.async_copy(x_hbm.at[sidx], xg, sem),
                    pltpu.async_copy(c_hbm.at[pl.ds(off, _CH)], cg, sem),
                ))
            ds = []
            for b, (sidx, didx, xg, cg, m, sem) in enumerate(slots):
                dg[b][0].wait()
                dg[b][1].wait()
                mul_rows(xg, cg, m)
                ds.append(pltpu.async_copy(m, aggsh.at[didx], sem, add=True))
            for d in ds:
                d.wait()
            return carry

        lax.fori_loop(0, nsup, super_it, 0)

        def tail(k, carry):
            off = wid * _EPW + (nsup * nbuf + k) * _CH
            sidx, didx, xg, cg, m, sem = slots[0]
            pltpu.sync_copy(src_hbm.at[pl.ds(off, _CH)], sidx)
            pltpu.sync_copy(dst_hbm.at[pl.ds(off, _CH)], didx)
            pltpu.async_copy(x_hbm.at[sidx], xg, sem).wait()
            pltpu.sync_copy(c_hbm.at[pl.ds(off, _CH)], cg)
            mul_rows(xg, cg, m)
            pltpu.sync_copy(m, aggsh.at[didx], add=True)
            return carry

        if ntail:
            lax.fori_loop(0, ntail, tail, 0)
        plsc.subcore_barrier()
        pltpu.sync_copy(aggsh.at[pl.ds(sid * _RPT, _RPT)],
                        out_hbm.at[pl.ds(cid * _N + sid * _RPT, _RPT)])

    return msg


_BE = 2000
_S3 = math.sqrt(3.0)
_S5 = math.sqrt(5.0)
_S15 = math.sqrt(15.0)
_RB_VALS = np.linspace(0.0, 5.0, 12)[1:-1]
_RB_STEP = float(_RB_VALS[1] - _RB_VALS[0])
_RB_C = 1.14136 * (math.e ** 2)
_SQRT_NB = math.sqrt(10.0)


def _coef_body(dps, *refs):
    vec_ref, pa_ref, pb_ref = refs[0:3]
    wrefs = refs[3:3 + 4 * len(dps)]
    orefs = refs[3 + 4 * len(dps):]

    v = vec_ref[...]
    r2 = jnp.sum(v * v, axis=1, keepdims=True) + 1e-9
    rinv = jax.lax.rsqrt(r2)
    r = r2 * rinv
    u = v * rinv
    li = lax.broadcasted_iota(jnp.int32, (1, 16), 1)
    up = jnp.where(li == 3, 1.0, u)
    qa = jnp.dot(up, pa_ref[...], preferred_element_type=jnp.float32)
    qb = jnp.dot(up, pb_ref[...], preferred_element_type=jnp.float32)
    q16 = qa * qb

    lif = li.astype(jnp.float32)
    diff = r * (1.0 / _RB_STEP) - (lif + 1.0)
    inside = jnp.abs(diff) < 1.0
    denom = jnp.maximum(jnp.where(inside, 1.0 - diff * diff, 1.0), 1e-6)
    emb = jnp.where(inside, _RB_C * jnp.exp(-1.0 / denom), 0.0) * _SQRT_NB

    for l in range(len(dps)):
        w1, b1, w2, wsheff = wrefs[4 * l:4 * l + 4]
        h = jnp.dot(emb, w1[...], preferred_element_type=jnp.float32) + b1[...]
        h = h * jax.nn.sigmoid(h)
        c = (jnp.dot(h, w2[...], preferred_element_type=jnp.float32)
             * jnp.dot(q16, wsheff[...], preferred_element_type=jnp.float32))
        hp = dps[l] // 2
        orefs[l][...] = _pack_tc(c[:, :hp], c[:, hp:])


_SH_A = np.zeros((9, 10), np.float32)
_SH_A[0, 0] = 1.0
_SH_A[1, 1] = _S3
_SH_A[2, 2] = _S3
_SH_A[3, 3] = _S3
_SH_A[4, 4] = _S15
_SH_A[5, 5] = _S15
_SH_A[6, 6] = _S5
_SH_A[6, 9] = -0.5 * _S5
_SH_A[6, 8] = -0.5 * _S5
_SH_A[7, 7] = _S15
_SH_A[8, 8] = 0.5 * _S15
_SH_A[8, 9] = -0.5 * _S15

_QA_IDX = [3, 0, 1, 2, 0, 0, 1, 1, 2, 0, 3, 3, 3, 3, 3, 3]
_QB_IDX = [3, 3, 3, 3, 2, 1, 1, 2, 2, 0, 3, 3, 3, 3, 3, 3]
_PA_NP = np.zeros((16, 16), np.float32)
_PB_NP = np.zeros((16, 16), np.float32)
for _m in range(16):
    _PA_NP[_QA_IDX[_m], _m] = 1.0
    _PB_NP[_QB_IDX[_m], _m] = 1.0


def _coef(vec, wlist, dps):
    flat_w = [w for tup in wlist for w in tup]
    full = lambda shape: pl.BlockSpec(shape, lambda i: (0, 0))
    pa = jnp.asarray(_PA_NP)
    pb = jnp.asarray(_PB_NP)
    return pl.pallas_call(
        functools.partial(_coef_body, tuple(dps)),
        grid=(_E // _BE,),
        in_specs=[pl.BlockSpec((_BE, 16), lambda i: (i, 0)),
                  full((16, 16)), full((16, 16))]
        + [full(w.shape) for w in flat_w],
        out_specs=[pl.BlockSpec((_BE, dp // 2), lambda i: (i, 0))
                   for dp in dps],
        out_shape=[jax.ShapeDtypeStruct((_E, dp // 2), jnp.int32)
                   for dp in dps],
    )(vec, pa, pb, *flat_w)


_BN = 2000


def _node_body(last, dpo, x_ref, na_ref, agg_ref, wsc_ref, wmsg_ref, o_ref,
               op_ref=None):
    xn = x_ref[...] * na_ref[...]
    sc = jnp.dot(xn, wsc_ref[...], preferred_element_type=jnp.float32)
    a = (agg_ref[0] + agg_ref[1]) * _INV_SQRT_NN
    out = sc + jnp.dot(a, wmsg_ref[...], preferred_element_type=jnp.float32)
    if not last:
        out = out * jax.nn.sigmoid(out)
        op_ref[...] = _pack_tc(out[:, :dpo // 2], out[:, dpo // 2:])
    o_ref[...] = out


def _node(xp, na, agg2, wsc, wmsg, last, dpo):
    dpi = xp.shape[1]
    agg3 = agg2.reshape(2, _N, dpi)
    out_specs = [pl.BlockSpec((_BN, dpo), lambda i: (i, 0))]
    out_shape = [jax.ShapeDtypeStruct((_N, dpo), jnp.float32)]
    if not last:
        out_specs.append(pl.BlockSpec((_BN, dpo // 2), lambda i: (i, 0)))
        out_shape.append(jax.ShapeDtypeStruct((_N, dpo // 2), jnp.int32))
    res = pl.pallas_call(
        functools.partial(_node_body, last, dpo),
        grid=(_N // _BN,),
        in_specs=[
            pl.BlockSpec((_BN, dpi), lambda i: (i, 0)),
            pl.BlockSpec((_BN, 1), lambda i: (i, 0)),
            pl.BlockSpec((2, _BN, dpi), lambda i: (0, i, 0)),
            pl.BlockSpec(wsc.shape, lambda i: (0, 0)),
            pl.BlockSpec(wmsg.shape, lambda i: (0, 0)),
        ],
        out_specs=out_specs,
        out_shape=out_shape,
    )(xp, na, agg3, wsc, wmsg)
    return res if not last else (res[0], None)


def _pack_body(x_ref, o_ref):
    x = x_ref[...]
    o_ref[...] = _pack_tc(x[:, :64], x[:, 64:])


def _pack128(x):
    return pl.pallas_call(
        _pack_body,
        grid=(_N // _BN,),
        in_specs=[pl.BlockSpec((_BN, 128), lambda i: (i, 0))],
        out_specs=pl.BlockSpec((_BN, 64), lambda i: (i, 0)),
        out_shape=jax.ShapeDtypeStruct((_N, 64), jnp.int32),
    )(x)


def _mix_body(a_ref, b_ref, o_ref, op_ref):
    x = 0.5 * (a_ref[...] + b_ref[...])
    o_ref[...] = x
    op_ref[...] = _pack_tc(x[:, :64], x[:, 64:])


def _mix_pack(a, b):
    return pl.pallas_call(
        _mix_body,
        grid=(_N // _BN,),
        in_specs=[pl.BlockSpec((_BN, 128), lambda i: (i, 0)),
                  pl.BlockSpec((_BN, 128), lambda i: (i, 0))],
        out_specs=[pl.BlockSpec((_BN, 128), lambda i: (i, 0)),
                   pl.BlockSpec((_BN, 64), lambda i: (i, 0))],
        out_shape=[jax.ShapeDtypeStruct((_N, 128), jnp.float32),
                   jax.ShapeDtypeStruct((_N, 64), jnp.int32)],
    )(a, b)


def _norm_body(x_ref, b_ref, o_ref):
    xo = x_ref[...]
    bt = b_ref[...]
    ss = jnp.sum(xo * xo, axis=1, keepdims=True)
    gids = lax.broadcasted_iota(jnp.int32, (1, _G), 1)
    oh = (bt == gids).astype(jnp.float32)
    g = jnp.sum(oh * ss, axis=0, keepdims=True)
    f = jnp.sqrt(g + 1e-12)
    fb = jnp.sum(oh * f, axis=1, keepdims=True)
    o_ref[...] = xo / fb


def _norm(xo, bt):
    return pl.pallas_call(
        _norm_body,
        in_specs=[
            pl.BlockSpec((_N, 128), lambda: (0, 0)),
            pl.BlockSpec((_N, 1), lambda: (0, 0)),
        ],
        out_specs=pl.BlockSpec((_N, 128), lambda: (0, 0)),
        out_shape=jax.ShapeDtypeStruct((_N, 128), jnp.float32),
    )(xo, bt)


def _padw(w, r, c):
    return jnp.zeros((r, c), jnp.float32).at[:w.shape[0], :w.shape[1]].set(w)


def _run_net(pos16, xin, xin_p, na, src, dst, layers):
    vec = _geom(pos16, src, dst)
    dpis = [128, 64, 64, 64]
    dpos = [64, 64, 64, 128]
    at = jnp.asarray(_SH_A.T)
    wlist = [
        (
            _padw(p['W1'], 16, 64),
            p['b1'].reshape(1, 64),
            _padw(p['W2'], 64, dpis[l]),
            _padw(at @ p['W_sh'], 16, dpis[l]),
        )
        for l, p in enumerate(layers)
    ]
    cs = _coef(vec, wlist, dpis)
    h, hpk = xin, xin_p
    for l, p in enumerate(layers):
        agg2 = _make_msg(dpis[l])(hpk, cs[l], src, dst)
        wsc = _padw(p['W_sc'], dpis[l], dpos[l])
        wmsg = _padw(p['W_msg'], dpis[l], dpos[l])
        h, hpk = _node(h, na, agg2, wsc, wmsg, last=(l == 3), dpo=dpos[l])
    return h


def kernel(x, pos, batch, edge_index, node_attr, x_final_state,
           pos_final_state, edge_index_final_state, params):
    f32 = jnp.float32
    src = edge_index[0].astype(jnp.int32)
    dst = edge_index[1].astype(jnp.int32)
    srcf = edge_index_final_state[0].astype(jnp.int32)
    dstf = edge_index_final_state[1].astype(jnp.int32)
    pos16 = jnp.zeros((_N, 16), f32).at[:, :3].set(pos)
    posf16 = jnp.zeros((_N, 16), f32).at[:, :3].set(pos_final_state)
    post16 = 0.5 * (pos16 + posf16)
    bt = batch.astype(jnp.int32).reshape(_N, 1)

    na = node_attr
    out_i = _norm(_run_net(pos16, x, _pack128(x), na, src, dst,
                           params['init']), bt)
    out_f = _norm(_run_net(posf16, x_final_state, _pack128(x_final_state),
                           na, srcf, dstf, params['final']), bt)
    x_ts, xts_p = _mix_pack(out_i, out_f)
    out_ts = _norm(_run_net(post16, x_ts, xts_p, na, src, dst,
                            params['interp']), bt)
    return out_ts
